# Initial kernel scaffold; baseline (speedup 1.0000x reference)
import jax
import jax.numpy as jnp
from jax.experimental import pallas as pl


def kernel(z, table):
    raise NotImplementedError("write your pallas kernel here")


# speedup vs baseline: 523.2651x; 523.2651x over previous
import functools

import jax
import jax.numpy as jnp
from jax import lax
from jax.experimental import pallas as pl
from jax.experimental.pallas import tpu as pltpu
from jax.experimental.pallas import tpu_sc as plsc

_LANES = 16
_NC = 2
_NS = 16
_NW = _NC * _NS

_CHUNK = 8192
_TBL_PAD = 128


def _build(n):
    per_w = n // _NW
    nchunk = per_w // _CHUNK
    vecs = _CHUNK // _LANES
    mesh = plsc.VectorSubcoreMesh(core_axis_name="c", subcore_axis_name="s")

    @functools.partial(
        pl.kernel,
        mesh=mesh,
        compiler_params=pltpu.CompilerParams(needs_layout_passes=False),
        out_type=jax.ShapeDtypeStruct((n,), jnp.float32),
        scratch_types=[
            pltpu.VMEM((_TBL_PAD,), jnp.float32),
            pltpu.VMEM((_CHUNK,), jnp.int32),
            pltpu.VMEM((_CHUNK,), jnp.float32),
            pltpu.SemaphoreType.DMA,
        ],
    )
    def run(z_hbm, tbl_hbm, out_hbm, tbl_v, z_v, o_v, sem):
        wid = lax.axis_index("s") * _NC + lax.axis_index("c")
        base = wid * per_w
        pltpu.sync_copy(tbl_hbm, tbl_v)
        for c in range(nchunk):
            off = base + c * _CHUNK
            pltpu.sync_copy(z_hbm.at[pl.ds(off, _CHUNK)], z_v)

            def body(i, carry):
                zv = z_v[pl.ds(i * _LANES, _LANES)]
                vals = plsc.load_gather(tbl_v, [zv - 1])
                o_v[pl.ds(i * _LANES, _LANES)] = vals
                return carry

            lax.fori_loop(0, vecs, body, 0)
            pltpu.sync_copy(o_v, out_hbm.at[pl.ds(off, _CHUNK)])

    return run


@jax.jit
def kernel(z, table):
    n = z.shape[0]
    tbl = jnp.pad(table.reshape(-1), (0, _TBL_PAD - table.shape[0]))
    return _build(n)(z.astype(jnp.int32), tbl)


# speedup vs baseline: 696.2245x; 1.3305x over previous
import functools

import jax
import jax.numpy as jnp
from jax import lax
from jax.experimental import pallas as pl
from jax.experimental.pallas import tpu as pltpu
from jax.experimental.pallas import tpu_sc as plsc

_LANES = 16
_NC = 2
_NS = 16
_NW = _NC * _NS

_CHUNK = 16384
_TBL_PAD = 128


def _build(n):
    per_w = n // _NW
    nchunk = per_w // _CHUNK
    vecs = _CHUNK // _LANES
    mesh = plsc.VectorSubcoreMesh(core_axis_name="c", subcore_axis_name="s")

    @functools.partial(
        pl.kernel,
        mesh=mesh,
        compiler_params=pltpu.CompilerParams(needs_layout_passes=False),
        out_type=jax.ShapeDtypeStruct((n,), jnp.float32),
        scratch_types=[
            pltpu.VMEM((_TBL_PAD,), jnp.float32),
            pltpu.VMEM((_CHUNK,), jnp.int32),
            pltpu.VMEM((_CHUNK,), jnp.int32),
            pltpu.VMEM((_CHUNK,), jnp.float32),
            pltpu.VMEM((_CHUNK,), jnp.float32),
            pltpu.SemaphoreType.DMA,
            pltpu.SemaphoreType.DMA,
            pltpu.SemaphoreType.DMA,
            pltpu.SemaphoreType.DMA,
        ],
    )
    def run(z_hbm, tbl_hbm, out_hbm, tbl_v, z0, z1, o0, o1,
            si0, si1, so0, so1):
        wid = lax.axis_index("s") * _NC + lax.axis_index("c")
        base = wid * per_w
        pltpu.sync_copy(tbl_hbm, tbl_v)

        zbuf = (z0, z1)
        obuf = (o0, o1)
        isem = (si0, si1)
        osem = (so0, so1)

        def in_copy(c):
            return pltpu.make_async_copy(
                z_hbm.at[pl.ds(base + c * _CHUNK, _CHUNK)],
                zbuf[c % 2], isem[c % 2])

        def out_copy(c):
            return pltpu.make_async_copy(
                obuf[c % 2],
                out_hbm.at[pl.ds(base + c * _CHUNK, _CHUNK)],
                osem[c % 2])

        in_copy(0).start()
        for c in range(nchunk):
            if c + 1 < nchunk:
                in_copy(c + 1).start()
            in_copy(c).wait()
            if c >= 2:
                out_copy(c - 2).wait()

            z_v = zbuf[c % 2]
            o_v = obuf[c % 2]

            def body(i, carry):
                zv = z_v[pl.ds(i * _LANES, _LANES)]
                vals = plsc.load_gather(tbl_v, [zv - 1])
                o_v[pl.ds(i * _LANES, _LANES)] = vals
                return carry

            lax.fori_loop(0, vecs, body, 0)
            out_copy(c).start()

        if nchunk >= 2:
            out_copy(nchunk - 2).wait()
        out_copy(nchunk - 1).wait()

    return run


@jax.jit
def kernel(z, table):
    n = z.shape[0]
    tbl = jnp.pad(table.reshape(-1), (0, _TBL_PAD - table.shape[0]))
    return _build(n)(z.astype(jnp.int32), tbl)


# speedup vs baseline: 1142.6625x; 1.6412x over previous
import functools

import jax
import jax.numpy as jnp
from jax import lax
from jax.experimental import pallas as pl
from jax.experimental.pallas import tpu as pltpu
from jax.experimental.pallas import tpu_sc as plsc

_LANES = 16
_NC = 2
_NS = 16
_NW = _NC * _NS

_CHUNK = 16384
_TBL_PAD = 128


def _build(n):
    per_w = n // _NW
    nchunk = per_w // _CHUNK
    vecs = _CHUNK // _LANES
    mesh = plsc.VectorSubcoreMesh(core_axis_name="c", subcore_axis_name="s")

    @functools.partial(
        pl.kernel,
        mesh=mesh,
        compiler_params=pltpu.CompilerParams(needs_layout_passes=False),
        out_type=jax.ShapeDtypeStruct((n,), jnp.float32),
        scratch_types=[
            pltpu.VMEM((_TBL_PAD,), jnp.float32),
            pltpu.VMEM((_CHUNK,), jnp.int32),
            pltpu.VMEM((_CHUNK,), jnp.int32),
            pltpu.VMEM((_CHUNK,), jnp.float32),
            pltpu.VMEM((_CHUNK,), jnp.float32),
            pltpu.SemaphoreType.DMA,
            pltpu.SemaphoreType.DMA,
            pltpu.SemaphoreType.DMA,
            pltpu.SemaphoreType.DMA,
        ],
    )
    def run(z_hbm, tbl_hbm, out_hbm, tbl_v, z0, z1, o0, o1,
            si0, si1, so0, so1):
        wid = lax.axis_index("s") * _NC + lax.axis_index("c")
        base = wid * per_w
        pltpu.sync_copy(tbl_hbm, tbl_v)

        zbuf = (z0, z1)
        obuf = (o0, o1)
        isem = (si0, si1)
        osem = (so0, so1)

        def in_copy(c):
            return pltpu.make_async_copy(
                z_hbm.at[pl.ds(base + c * _CHUNK, _CHUNK)],
                zbuf[c % 2], isem[c % 2])

        def out_copy(c):
            return pltpu.make_async_copy(
                obuf[c % 2],
                out_hbm.at[pl.ds(base + c * _CHUNK, _CHUNK)],
                osem[c % 2])

        in_copy(0).start()
        for c in range(nchunk):
            if c + 1 < nchunk:
                in_copy(c + 1).start()
            in_copy(c).wait()
            if c >= 2:
                out_copy(c - 2).wait()

            z_v = zbuf[c % 2]
            o_v = obuf[c % 2]

            @plsc.parallel_loop(0, _CHUNK, _LANES, unroll=8)
            def _(i):
                zv = z_v[pl.ds(i, _LANES)]
                o_v[pl.ds(i, _LANES)] = plsc.load_gather(tbl_v, [zv - 1])

            out_copy(c).start()

        if nchunk >= 2:
            out_copy(nchunk - 2).wait()
        out_copy(nchunk - 1).wait()

    return run


@jax.jit
def kernel(z, table):
    n = z.shape[0]
    tbl = jnp.pad(table.reshape(-1), (0, _TBL_PAD - table.shape[0]))
    return _build(n)(z.astype(jnp.int32), tbl)
